# baseline (speedup 1.0000x reference)
import functools

import jax
import jax.numpy as jnp
from jax import lax
from jax.experimental import pallas as pl
from jax.experimental.pallas import tpu as pltpu
from jax.experimental.pallas import tpu_sc as plsc

_N = 128
_M = _N * (_N + 1) // 2
_NC = 2
_NS = 16
_NW = _NC * _NS
_L = 16
_NB = _N // _L


def _sc_body(x_hbm, o_hbm, xv00, xv01, xv10, xv11, ov0, ov1,
             isem0, isem1, osem0, osem1, npairs):
    wid = lax.axis_index("s") * _NC + lax.axis_index("c")
    s0 = wid * (2 * npairs)
    xvs = ((xv00, xv01), (xv10, xv11))
    ovs = (ov0, ov1)
    isems = (isem0, isem1)
    osems = (osem0, osem1)

    jvs = [lax.iota(jnp.int32, _L) + _L * k for k in range(_NB)]
    svs = [127 * j - ((j * (j - 1)) >> 1) for j in jvs]

    def in_copies(pp, q):
        b = s0 + 2 * pp
        return (pltpu.make_async_copy(x_hbm.at[b], xvs[q][0], isems[q]),
                pltpu.make_async_copy(x_hbm.at[b + 1], xvs[q][1], isems[q]))

    def out_copy(pp, q):
        return pltpu.make_async_copy(
            ovs[q], o_hbm.at[pl.ds(s0 + 2 * pp, 2)], osems[q])

    def expand_pair(q):
        ov = ovs[q]

        for rb in range(_NB):
            def row_body(r, carry, rb=rb):
                sr = 127 * r - ((r * (r - 1)) >> 1)
                stores = []
                for samp in range(2):
                    xv = xvs[q][samp]
                    for k2 in range(rb, _NB):
                        stores.append((samp, _L * k2,
                                       xv[pl.ds(sr + _L * k2, _L)]))
                    bidx = [svs[k] + r for k in range(rb)]
                    bidx.append(jnp.where(jvs[rb] < r, svs[rb] + r,
                                          sr + jvs[rb]))
                    for k in range(rb + 1):
                        stores.append((samp, _L * k,
                                       plsc.load_gather(xv, [bidx[k]])))
                for samp, col, val in stores:
                    ov[samp, r, pl.ds(col, _L)] = val
                return carry

            lax.fori_loop(_L * rb, _L * (rb + 1), row_body, 0, unroll=4)

    for cp in in_copies(0, 0) + in_copies(1, 1):
        cp.start()

    def step_body(p, carry):
        for q in range(2):
            pp = 2 * p + q
            for cp in in_copies(pp, q):
                cp.wait()

            @pl.when(p >= 1)
            def _wait_out():
                out_copy(pp - 2, q).wait()

            expand_pair(q)
            out_copy(pp, q).start()

            @pl.when(pp < npairs - 2)
            def _next_in():
                for cp in in_copies(pp + 2, q):
                    cp.start()

        return carry

    lax.fori_loop(0, npairs // 2, step_body, 0, unroll=False)
    out_copy(npairs - 2, 0).wait()
    out_copy(npairs - 1, 1).wait()


def kernel(input):
    b = input.shape[0]
    assert input.shape[1] == _M and b % (4 * _NW) == 0
    npairs = b // (2 * _NW)
    mesh = plsc.VectorSubcoreMesh(
        core_axis_name="c", subcore_axis_name="s",
        num_cores=_NC, num_subcores=_NS)
    return pl.kernel(
        functools.partial(_sc_body, npairs=npairs),
        out_type=jax.ShapeDtypeStruct((b, _N, _N), input.dtype),
        mesh=mesh,
        scratch_types=[
            pltpu.VMEM((_M,), jnp.float32),
            pltpu.VMEM((_M,), jnp.float32),
            pltpu.VMEM((_M,), jnp.float32),
            pltpu.VMEM((_M,), jnp.float32),
            pltpu.VMEM((2, _N, _N), jnp.float32),
            pltpu.VMEM((2, _N, _N), jnp.float32),
            pltpu.SemaphoreType.DMA,
            pltpu.SemaphoreType.DMA,
            pltpu.SemaphoreType.DMA,
            pltpu.SemaphoreType.DMA,
        ],
        compiler_params=pltpu.CompilerParams(needs_layout_passes=False),
    )(input)


# speedup vs baseline: 1.0270x; 1.0270x over previous
import functools

import jax
import jax.numpy as jnp
from jax import lax
from jax.experimental import pallas as pl
from jax.experimental.pallas import tpu as pltpu
from jax.experimental.pallas import tpu_sc as plsc

_N = 128
_M = _N * (_N + 1) // 2
_NC = 2
_NS = 16
_NW = _NC * _NS
_L = 16
_NB = _N // _L


def _sc_body(x_hbm, o_hbm, xv00, xv01, xv10, xv11, ov0, ov1,
             isem0, isem1, osem0, osem1, npairs):
    wid = lax.axis_index("s") * _NC + lax.axis_index("c")
    s0 = wid * (2 * npairs)
    xvs = ((xv00, xv01), (xv10, xv11))
    ovs = (ov0, ov1)
    isems = (isem0, isem1)
    osems = (osem0, osem1)

    jvs = [lax.iota(jnp.int32, _L) + _L * k for k in range(_NB)]
    svs = [127 * j - ((j * (j - 1)) >> 1) for j in jvs]

    def in_copies(pp, q):
        b = s0 + 2 * pp
        return (pltpu.make_async_copy(x_hbm.at[b], xvs[q][0], isems[q]),
                pltpu.make_async_copy(x_hbm.at[b + 1], xvs[q][1], isems[q]))

    def out_copy(pp, q):
        return pltpu.make_async_copy(
            ovs[q], o_hbm.at[pl.ds(s0 + 2 * pp, 2)], osems[q])

    def expand_pair(q):
        ov = ovs[q]

        for rb in range(_NB):
            def row_body(r, carry, rb=rb):
                sr = 127 * r - ((r * (r - 1)) >> 1)
                stores = []
                for samp in range(2):
                    xv = xvs[q][samp]
                    for k2 in range(rb, _NB):
                        stores.append((samp, _L * k2,
                                       xv[pl.ds(sr + _L * k2, _L)]))
                    bidx = [svs[k] + r for k in range(rb)]
                    bidx.append(jnp.where(jvs[rb] < r, svs[rb] + r,
                                          sr + jvs[rb]))
                    for k in range(rb + 1):
                        stores.append((samp, _L * k,
                                       plsc.load_gather(xv, [bidx[k]])))
                for samp, col, val in stores:
                    ov[samp, r, pl.ds(col, _L)] = val
                return carry

            lax.fori_loop(_L * rb, _L * (rb + 1), row_body, 0, unroll=2)

    for cp in in_copies(0, 0) + in_copies(1, 1):
        cp.start()

    def step_body(p, carry):
        for q in range(2):
            pp = 2 * p + q
            for cp in in_copies(pp, q):
                cp.wait()

            @pl.when(p >= 1)
            def _wait_out():
                out_copy(pp - 2, q).wait()

            expand_pair(q)
            out_copy(pp, q).start()

            @pl.when(pp < npairs - 2)
            def _next_in():
                for cp in in_copies(pp + 2, q):
                    cp.start()

        return carry

    lax.fori_loop(0, npairs // 2, step_body, 0, unroll=False)
    out_copy(npairs - 2, 0).wait()
    out_copy(npairs - 1, 1).wait()


def kernel(input):
    b = input.shape[0]
    assert input.shape[1] == _M and b % (4 * _NW) == 0
    npairs = b // (2 * _NW)
    mesh = plsc.VectorSubcoreMesh(
        core_axis_name="c", subcore_axis_name="s",
        num_cores=_NC, num_subcores=_NS)
    return pl.kernel(
        functools.partial(_sc_body, npairs=npairs),
        out_type=jax.ShapeDtypeStruct((b, _N, _N), input.dtype),
        mesh=mesh,
        scratch_types=[
            pltpu.VMEM((_M,), jnp.float32),
            pltpu.VMEM((_M,), jnp.float32),
            pltpu.VMEM((_M,), jnp.float32),
            pltpu.VMEM((_M,), jnp.float32),
            pltpu.VMEM((2, _N, _N), jnp.float32),
            pltpu.VMEM((2, _N, _N), jnp.float32),
            pltpu.SemaphoreType.DMA,
            pltpu.SemaphoreType.DMA,
            pltpu.SemaphoreType.DMA,
            pltpu.SemaphoreType.DMA,
        ],
        compiler_params=pltpu.CompilerParams(needs_layout_passes=False),
    )(input)
